# Initial kernel scaffold; baseline (speedup 1.0000x reference)
import functools

import jax
import jax.numpy as jnp
from jax import lax
from jax.experimental import pallas as pl
from jax.experimental.pallas import tpu as pltpu
from jax.experimental.pallas import tpu_sc as plsc

N_TOK = 4096
D_DIM = 1024
H_DIM = 4096
N_EXP = 8
TOPK = 2
N_ASSIGN = N_TOK * TOPK

BLK = 512
L_PAD = N_ASSIGN + N_EXP * BLK
NB = L_PAD // BLK
HT = 512
NHT = H_DIM // HT

_GW = 32


def _router_body(x_ref, wg_ref, bg_ref, out_ref):
    logits = jnp.dot(x_ref[...], wg_ref[...],
                     preferred_element_type=jnp.float32) + bg_ref[...]
    lanes = lax.broadcasted_iota(jnp.int32, logits.shape, 1)
    m1 = jnp.max(logits, axis=1, keepdims=True)
    i1 = jnp.min(jnp.where(logits == m1, lanes, 128), axis=1, keepdims=True)
    l2 = jnp.where(lanes == i1, -jnp.inf, logits)
    m2 = jnp.max(l2, axis=1, keepdims=True)
    i2 = jnp.min(jnp.where(l2 == m2, lanes, 128), axis=1, keepdims=True)
    w0 = jax.nn.sigmoid(m1 - m2)
    w1 = 1.0 - w0
    out = jnp.where(lanes == 0, w0,
          jnp.where(lanes == 1, w1,
          jnp.where(lanes == 2, i1.astype(jnp.float32),
          jnp.where(lanes == 3, i2.astype(jnp.float32), 0.0))))
    out_ref[...] = out


def _router(xf, wg_pad, bg_pad):
    blk = 512
    return pl.pallas_call(
        _router_body,
        grid=(N_TOK // blk,),
        in_specs=[
            pl.BlockSpec((blk, D_DIM), lambda i: (i, 0)),
            pl.BlockSpec((D_DIM, 128), lambda i: (0, 0)),
            pl.BlockSpec((1, 128), lambda i: (0, 0)),
        ],
        out_specs=pl.BlockSpec((blk, 128), lambda i: (i, 0)),
        out_shape=jax.ShapeDtypeStruct((N_TOK, 128), jnp.float32),
    )(xf, wg_pad, bg_pad)


def _sc_gather(table, idx):
    n = idx.shape[0]
    d = table.shape[1]
    idx2 = idx.reshape(1, n)
    mesh = plsc.VectorSubcoreMesh(core_axis_name="c", subcore_axis_name="s")

    @functools.partial(
        pl.kernel,
        out_type=jax.ShapeDtypeStruct((n, d), table.dtype),
        mesh=mesh,
    )
    def k(x_hbm, i_hbm, o_hbm):
        def body(i_vmem, o_vmem):
            pltpu.sync_copy(x_hbm.at[i_vmem.at[0]], o_vmem)

        pltpu.emit_pipeline(
            body,
            grid=(n // _GW,),
            in_specs=[pl.BlockSpec((1, _GW), index_map=lambda i: (0, i))],
            out_specs=[pl.BlockSpec((_GW, d), index_map=lambda i: (i, 0))],
            core_axis_name=("c", "s"),
            dimension_semantics=(pltpu.PARALLEL,),
        )(i_hbm, o_hbm)

    return k(table, idx2)


def _ffn_body(beid_ref, nblk_ref, xs_ref, w1_ref, b1_ref, w2_ref, b2_ref,
              ys_ref):
    b = pl.program_id(0)
    ht = pl.program_id(1)

    @pl.when(b < nblk_ref[0])
    def _():
        h = jnp.dot(xs_ref[...], w1_ref[0],
                    preferred_element_type=jnp.float32) + b1_ref[...]
        h = jax.nn.gelu(h)
        contrib = jnp.dot(h, w2_ref[0], preferred_element_type=jnp.float32)

        @pl.when(ht == 0)
        def _():
            ys_ref[...] = contrib + b2_ref[...]

        @pl.when(ht != 0)
        def _():
            ys_ref[...] += contrib


def _grouped_ffn(beid, nblk, xs, W1, b1, W2, b2):
    grid_spec = pltpu.PrefetchScalarGridSpec(
        num_scalar_prefetch=2,
        grid=(NB, NHT),
        in_specs=[
            pl.BlockSpec((BLK, D_DIM), lambda b, ht, beid, nblk: (b, 0)),
            pl.BlockSpec((1, D_DIM, HT),
                         lambda b, ht, beid, nblk: (beid[b], 0, ht)),
            pl.BlockSpec((1, HT), lambda b, ht, beid, nblk: (beid[b], ht)),
            pl.BlockSpec((1, HT, D_DIM),
                         lambda b, ht, beid, nblk: (beid[b], ht, 0)),
            pl.BlockSpec((1, D_DIM), lambda b, ht, beid, nblk: (beid[b], 0)),
        ],
        out_specs=pl.BlockSpec(
            (BLK, D_DIM),
            lambda b, ht, beid, nblk: (jnp.where(b < nblk[0], b, NB - 1), 0)),
    )
    return pl.pallas_call(
        _ffn_body,
        grid_spec=grid_spec,
        out_shape=jax.ShapeDtypeStruct((L_PAD, D_DIM), jnp.float32),
        compiler_params=pltpu.CompilerParams(
            dimension_semantics=("arbitrary", "arbitrary")),
    )(beid, nblk, xs, W1, b1, W2, b2)


def _ln_body(g0_ref, g1_ref, w0_ref, w1_ref, gamma_ref, beta_ref, out_ref):
    y = (w0_ref[:, 0:1] * g0_ref[...] + w1_ref[:, 0:1] * g1_ref[...])
    mu = jnp.mean(y, axis=1, keepdims=True)
    yc = y - mu
    var = jnp.mean(yc * yc, axis=1, keepdims=True)
    out_ref[...] = (yc * lax.rsqrt(var + 1e-5) * gamma_ref[...]
                    + beta_ref[...])


def _combine_ln(g, w0b, w1b, gamma2, beta2):
    blk = 512
    nblk = N_TOK // blk
    return pl.pallas_call(
        _ln_body,
        grid=(nblk,),
        in_specs=[
            pl.BlockSpec((blk, D_DIM), lambda i: (i, 0)),
            pl.BlockSpec((blk, D_DIM), lambda i: (i + nblk, 0)),
            pl.BlockSpec((blk, 128), lambda i: (i, 0)),
            pl.BlockSpec((blk, 128), lambda i: (i, 0)),
            pl.BlockSpec((1, D_DIM), lambda i: (0, 0)),
            pl.BlockSpec((1, D_DIM), lambda i: (0, 0)),
        ],
        out_specs=pl.BlockSpec((blk, D_DIM), lambda i: (i, 0)),
        out_shape=jax.ShapeDtypeStruct((N_TOK, D_DIM), jnp.float32),
    )(g, g, w0b, w1b, gamma2, beta2)


def kernel(x, Wg, bg, W1, b1, W2, b2, gamma, beta):
    B, S, D = x.shape
    xf = x.reshape(B * S, D)

    wg_pad = jnp.zeros((D_DIM, 128), jnp.float32).at[:, :N_EXP].set(Wg)
    bg_pad = (jnp.full((128,), -1e30, jnp.float32).at[:N_EXP].set(bg)
              .reshape(1, 128))
    packed = _router(xf, wg_pad, bg_pad)

    w01 = packed[:, 0:TOPK]
    e01 = packed[:, TOPK:2 * TOPK].astype(jnp.int32)

    e_flat = e01.reshape(-1)
    t_flat = (jnp.arange(N_ASSIGN, dtype=jnp.int32) // TOPK)

    order = jnp.argsort(e_flat, stable=True)
    e_s = e_flat[order]
    t_s = t_flat[order]

    counts = jnp.bincount(e_flat, length=N_EXP)
    starts = jnp.cumsum(counts) - counts
    cp = ((counts + BLK - 1) // BLK) * BLK
    bounds = jnp.cumsum(cp)
    off = bounds - cp

    r = jnp.arange(N_ASSIGN, dtype=jnp.int32)
    p_s = (off[e_s] + (r - starts[e_s])).astype(jnp.int32)

    tok_pad = jnp.zeros((L_PAD,), jnp.int32).at[p_s].set(t_s)
    pos = jnp.zeros((N_ASSIGN,), jnp.int32).at[order].set(p_s)
    poscat = jnp.concatenate([pos[0::2], pos[1::2]])

    nblk_tot = (bounds[-1] // BLK).astype(jnp.int32)
    barange = jnp.arange(NB, dtype=jnp.int32)
    beid_raw = jnp.searchsorted(bounds, barange * BLK,
                                side="right").astype(jnp.int32)
    beid = jnp.where(barange < nblk_tot,
                     jnp.minimum(beid_raw, N_EXP - 1), e_s[-1])
    nblk1 = nblk_tot.reshape(1)

    xs = _sc_gather(xf, tok_pad)
    ys = _grouped_ffn(beid, nblk1, xs, W1, b1, W2, b2)
    g = _sc_gather(ys, poscat)

    w0b = jnp.broadcast_to(w01[:, 0:1], (N_TOK, 128))
    w1b = jnp.broadcast_to(w01[:, 1:2], (N_TOK, 128))
    out = _combine_ln(g, w0b, w1b, gamma.reshape(1, D_DIM),
                      beta.reshape(1, D_DIM))
    return out.reshape(B, S, D)


# speedup vs baseline: 1.0942x; 1.0942x over previous
import functools

import jax
import jax.numpy as jnp
from jax import lax
from jax.experimental import pallas as pl
from jax.experimental.pallas import tpu as pltpu
from jax.experimental.pallas import tpu_sc as plsc

N_TOK = 4096
D_DIM = 1024
H_DIM = 4096
N_EXP = 8
TOPK = 2
N_ASSIGN = N_TOK * TOPK

BLK = 512
L_PAD = N_ASSIGN + N_EXP * BLK
NB = L_PAD // BLK
HT = 512
NHT = H_DIM // HT

_GW = 128
_SUB = D_DIM // 128


def _router_body(x_ref, wg_ref, bg_ref, out_ref):
    logits = jnp.dot(x_ref[...], wg_ref[...],
                     preferred_element_type=jnp.float32) + bg_ref[...]
    lanes = lax.broadcasted_iota(jnp.int32, logits.shape, 1)
    m1 = jnp.max(logits, axis=1, keepdims=True)
    i1 = jnp.min(jnp.where(logits == m1, lanes, 128), axis=1, keepdims=True)
    l2 = jnp.where(lanes == i1, -jnp.inf, logits)
    m2 = jnp.max(l2, axis=1, keepdims=True)
    i2 = jnp.min(jnp.where(l2 == m2, lanes, 128), axis=1, keepdims=True)
    w0 = jax.nn.sigmoid(m1 - m2)
    w1 = 1.0 - w0
    out = jnp.where(lanes == 0, w0,
          jnp.where(lanes == 1, w1,
          jnp.where(lanes == 2, i1.astype(jnp.float32),
          jnp.where(lanes == 3, i2.astype(jnp.float32), 0.0))))
    out_ref[...] = out


def _router(xf, wg_pad, bg_pad):
    blk = 512
    return pl.pallas_call(
        _router_body,
        grid=(N_TOK // blk,),
        in_specs=[
            pl.BlockSpec((blk, D_DIM), lambda i: (i, 0)),
            pl.BlockSpec((D_DIM, 128), lambda i: (0, 0)),
            pl.BlockSpec((1, 128), lambda i: (0, 0)),
        ],
        out_specs=pl.BlockSpec((blk, 128), lambda i: (i, 0)),
        out_shape=jax.ShapeDtypeStruct((N_TOK, 128), jnp.float32),
    )(xf, wg_pad, bg_pad)


def _sc_gather(table, idx):
    nrow = idx.shape[0]
    table = table.reshape(table.shape[0] * _SUB, 128)
    idx = (idx[:, None] * _SUB
           + jnp.arange(_SUB, dtype=jnp.int32)[None, :]).reshape(-1)
    n = nrow * _SUB
    d = 128
    idx2 = idx.reshape(1, n)
    mesh = plsc.VectorSubcoreMesh(core_axis_name="c", subcore_axis_name="s")

    @functools.partial(
        pl.kernel,
        out_type=jax.ShapeDtypeStruct((n, d), table.dtype),
        mesh=mesh,
    )
    def k(x_hbm, i_hbm, o_hbm):
        def body(i_vmem, o_vmem):
            pltpu.sync_copy(x_hbm.at[i_vmem.at[0]], o_vmem)

        pltpu.emit_pipeline(
            body,
            grid=(n // _GW,),
            in_specs=[pl.BlockSpec((1, _GW), index_map=lambda i: (0, i))],
            out_specs=[pl.BlockSpec((_GW, d), index_map=lambda i: (i, 0))],
            core_axis_name=("c", "s"),
            dimension_semantics=(pltpu.PARALLEL,),
        )(i_hbm, o_hbm)

    return k(table, idx2).reshape(nrow, D_DIM)


def _ffn_body(beid_ref, nblk_ref, xs_ref, w1_ref, b1_ref, w2_ref, b2_ref,
              ys_ref):
    b = pl.program_id(0)
    ht = pl.program_id(1)

    @pl.when(b < nblk_ref[0])
    def _():
        h = jnp.dot(xs_ref[...], w1_ref[0],
                    preferred_element_type=jnp.float32) + b1_ref[0]
        h = jax.nn.gelu(h)
        contrib = jnp.dot(h, w2_ref[0], preferred_element_type=jnp.float32)

        @pl.when(ht == 0)
        def _():
            ys_ref[...] = contrib + b2_ref[0]

        @pl.when(ht != 0)
        def _():
            ys_ref[...] += contrib


def _grouped_ffn(beid, nblk, xs, W1, b1, W2, b2):
    grid_spec = pltpu.PrefetchScalarGridSpec(
        num_scalar_prefetch=2,
        grid=(NB, NHT),
        in_specs=[
            pl.BlockSpec((BLK, D_DIM), lambda b, ht, beid, nblk: (b, 0)),
            pl.BlockSpec((1, D_DIM, HT),
                         lambda b, ht, beid, nblk: (beid[b], 0, ht)),
            pl.BlockSpec((1, 1, HT), lambda b, ht, beid, nblk: (beid[b], 0, ht)),
            pl.BlockSpec((1, HT, D_DIM),
                         lambda b, ht, beid, nblk: (beid[b], ht, 0)),
            pl.BlockSpec((1, 1, D_DIM),
                         lambda b, ht, beid, nblk: (beid[b], 0, 0)),
        ],
        out_specs=pl.BlockSpec(
            (BLK, D_DIM),
            lambda b, ht, beid, nblk: (jnp.where(b < nblk[0], b, NB - 1), 0)),
    )
    return pl.pallas_call(
        _ffn_body,
        grid_spec=grid_spec,
        out_shape=jax.ShapeDtypeStruct((L_PAD, D_DIM), jnp.float32),
        compiler_params=pltpu.CompilerParams(
            dimension_semantics=("arbitrary", "arbitrary")),
    )(beid, nblk, xs, W1, b1.reshape(N_EXP, 1, H_DIM), W2,
      b2.reshape(N_EXP, 1, D_DIM))


def _ln_body(g0_ref, g1_ref, w0_ref, w1_ref, gamma_ref, beta_ref, out_ref):
    y = (w0_ref[:, 0:1] * g0_ref[...] + w1_ref[:, 0:1] * g1_ref[...])
    mu = jnp.mean(y, axis=1, keepdims=True)
    yc = y - mu
    var = jnp.mean(yc * yc, axis=1, keepdims=True)
    out_ref[...] = (yc * lax.rsqrt(var + 1e-5) * gamma_ref[...]
                    + beta_ref[...])


def _combine_ln(g, w0b, w1b, gamma2, beta2):
    blk = 512
    nblk = N_TOK // blk
    return pl.pallas_call(
        _ln_body,
        grid=(nblk,),
        in_specs=[
            pl.BlockSpec((blk, D_DIM), lambda i: (i, 0)),
            pl.BlockSpec((blk, D_DIM), lambda i: (i + nblk, 0)),
            pl.BlockSpec((blk, 128), lambda i: (i, 0)),
            pl.BlockSpec((blk, 128), lambda i: (i, 0)),
            pl.BlockSpec((1, D_DIM), lambda i: (0, 0)),
            pl.BlockSpec((1, D_DIM), lambda i: (0, 0)),
        ],
        out_specs=pl.BlockSpec((blk, D_DIM), lambda i: (i, 0)),
        out_shape=jax.ShapeDtypeStruct((N_TOK, D_DIM), jnp.float32),
    )(g, g, w0b, w1b, gamma2, beta2)


def kernel(x, Wg, bg, W1, b1, W2, b2, gamma, beta):
    B, S, D = x.shape
    xf = x.reshape(B * S, D)

    wg_pad = jnp.zeros((D_DIM, 128), jnp.float32).at[:, :N_EXP].set(Wg)
    bg_pad = (jnp.full((128,), -1e30, jnp.float32).at[:N_EXP].set(bg)
              .reshape(1, 128))
    packed = _router(xf, wg_pad, bg_pad)

    w01 = packed[:, 0:TOPK]
    e01 = packed[:, TOPK:2 * TOPK].astype(jnp.int32)

    e_flat = e01.reshape(-1)
    t_flat = (jnp.arange(N_ASSIGN, dtype=jnp.int32) // TOPK)

    order = jnp.argsort(e_flat, stable=True)
    e_s = e_flat[order]
    t_s = t_flat[order]

    counts = jnp.bincount(e_flat, length=N_EXP)
    starts = jnp.cumsum(counts) - counts
    cp = ((counts + BLK - 1) // BLK) * BLK
    bounds = jnp.cumsum(cp)
    off = bounds - cp

    r = jnp.arange(N_ASSIGN, dtype=jnp.int32)
    p_s = (off[e_s] + (r - starts[e_s])).astype(jnp.int32)

    tok_pad = jnp.zeros((L_PAD,), jnp.int32).at[p_s].set(t_s)
    pos = jnp.zeros((N_ASSIGN,), jnp.int32).at[order].set(p_s)
    poscat = jnp.concatenate([pos[0::2], pos[1::2]])

    nblk_tot = (bounds[-1] // BLK).astype(jnp.int32)
    barange = jnp.arange(NB, dtype=jnp.int32)
    beid_raw = jnp.searchsorted(bounds, barange * BLK,
                                side="right").astype(jnp.int32)
    beid = jnp.where(barange < nblk_tot,
                     jnp.minimum(beid_raw, N_EXP - 1), e_s[-1])
    nblk1 = nblk_tot.reshape(1)

    xs = _sc_gather(xf, tok_pad)
    ys = _grouped_ffn(beid, nblk1, xs, W1, b1, W2, b2)
    g = _sc_gather(ys, poscat)

    w0b = jnp.broadcast_to(w01[:, 0:1], (N_TOK, 128))
    w1b = jnp.broadcast_to(w01[:, 1:2], (N_TOK, 128))
    out = _combine_ln(g, w0b, w1b, gamma.reshape(1, D_DIM),
                      beta.reshape(1, D_DIM))
    return out.reshape(B, S, D)
